# baseline (speedup 1.0000x reference)
import jax
import jax.numpy as jnp
from jax.experimental import pallas as pl
from jax.experimental.pallas import tpu as pltpu

Q = 1024
D = 1024
K = 16384
BK = 1024
NBLK = K // BK


def _patchcore_kernel(q_ref, m_ref, dist_ref, score_ref, acc_ref, qt_ref):
    k = pl.program_id(0)

    @pl.when(k == 0)
    def _():
        qt_ref[...] = q_ref[...].T

    m = m_ref[...]
    g = jax.lax.dot_general(
        m, qt_ref[...], (((1,), (0,)), ((), ())),
        preferred_element_type=jnp.float32)
    m_sq_half = 0.5 * jnp.sum(m * m, axis=1)
    part = jnp.min(m_sq_half[:, None] - g, axis=0)[None, :]

    @pl.when(k == 0)
    def _():
        acc_ref[...] = part

    @pl.when(k > 0)
    def _():
        acc_ref[...] = jnp.minimum(acc_ref[...], part)

    @pl.when(k == NBLK - 1)
    def _():
        qt = qt_ref[...]
        q_sq = jnp.sum(qt * qt, axis=0)[None, :]
        d2 = 2.0 * acc_ref[...] + q_sq
        dist = jnp.sqrt(jnp.maximum(d2, 1e-12))
        dist_ref[...] = dist
        score_ref[...] = jnp.max(dist, axis=1, keepdims=True)


@jax.jit
def kernel(queries, memory_bank):
    dist, score = pl.pallas_call(
        _patchcore_kernel,
        grid=(NBLK,),
        in_specs=[
            pl.BlockSpec((Q, D), lambda k: (0, 0)),
            pl.BlockSpec((BK, D), lambda k: (k, 0)),
        ],
        out_specs=[
            pl.BlockSpec((1, Q), lambda k: (0, 0)),
            pl.BlockSpec((1, 1), lambda k: (0, 0)),
        ],
        out_shape=[
            jax.ShapeDtypeStruct((1, Q), jnp.float32),
            jax.ShapeDtypeStruct((1, 1), jnp.float32),
        ],
        scratch_shapes=[
            pltpu.VMEM((1, Q), jnp.float32),
            pltpu.VMEM((D, Q), jnp.float32),
        ],
    )(queries, memory_bank)
    patch_scores = dist.reshape(Q)
    anomaly_map = patch_scores.reshape(32, 32)
    image_score = score.reshape(())
    return patch_scores, anomaly_map, image_score


# speedup vs baseline: 1.0329x; 1.0329x over previous
import jax
import jax.numpy as jnp
from jax.experimental import pallas as pl
from jax.experimental.pallas import tpu as pltpu

Q = 1024
D = 1024
K = 16384
BK = 1024
NBLK = K // BK


def _patchcore_kernel(q_ref, m_ref, dist_ref, score_ref, acc_ref, qt_ref):
    k = pl.program_id(0)

    @pl.when(k == 0)
    def _():
        qt_ref[...] = q_ref[...].T.astype(jnp.bfloat16)

    m = m_ref[...]
    g = jax.lax.dot_general(
        m.astype(jnp.bfloat16), qt_ref[...], (((1,), (0,)), ((), ())),
        preferred_element_type=jnp.float32)
    m_sq_half = 0.5 * jnp.sum(m * m, axis=1)
    part = jnp.min(m_sq_half[:, None] - g, axis=0)[None, :]

    @pl.when(k == 0)
    def _():
        acc_ref[...] = part

    @pl.when(k > 0)
    def _():
        acc_ref[...] = jnp.minimum(acc_ref[...], part)

    @pl.when(k == NBLK - 1)
    def _():
        q = q_ref[...]
        q_sq = jnp.sum(q * q, axis=1)[None, :]
        d2 = 2.0 * acc_ref[...] + q_sq
        dist = jnp.sqrt(jnp.maximum(d2, 1e-12))
        dist_ref[...] = dist
        score_ref[...] = jnp.max(dist, axis=1, keepdims=True)


@jax.jit
def kernel(queries, memory_bank):
    dist, score = pl.pallas_call(
        _patchcore_kernel,
        grid=(NBLK,),
        in_specs=[
            pl.BlockSpec((Q, D), lambda k: (0, 0)),
            pl.BlockSpec((BK, D), lambda k: (k, 0)),
        ],
        out_specs=[
            pl.BlockSpec((1, Q), lambda k: (0, 0)),
            pl.BlockSpec((1, 1), lambda k: (0, 0)),
        ],
        out_shape=[
            jax.ShapeDtypeStruct((1, Q), jnp.float32),
            jax.ShapeDtypeStruct((1, 1), jnp.float32),
        ],
        scratch_shapes=[
            pltpu.VMEM((1, Q), jnp.float32),
            pltpu.VMEM((D, Q), jnp.bfloat16),
        ],
    )(queries, memory_bank)
    patch_scores = dist.reshape(Q)
    anomaly_map = patch_scores.reshape(32, 32)
    image_score = score.reshape(())
    return patch_scores, anomaly_map, image_score
